# baseline (speedup 1.0000x reference)
import functools

import jax
import jax.numpy as jnp
from jax import lax
from jax.experimental import pallas as pl
from jax.experimental.pallas import tpu as pltpu
from jax.experimental.pallas import tpu_sc as plsc

N, E, F_IN, H, ED, G = 10000, 320000, 128, 64, 4, 64

NW = 32
EPW = E // NW
C = 80
NCH = EPW // C
RA = 624
TAIL = N - 16 * RA
WBC = 208


def _make_sc_agg(TW, W):
    mesh = plsc.VectorSubcoreMesh(core_axis_name="c", subcore_axis_name="s")

    @functools.partial(
        pl.kernel,
        mesh=mesh,
        out_type=jax.ShapeDtypeStruct((2, N, TW), jnp.float32),
        scratch_types=[
            pltpu.VMEM((C,), jnp.int32),
            pltpu.VMEM((C,), jnp.int32),
            pltpu.VMEM((C,), jnp.int32),
            pltpu.VMEM((C,), jnp.int32),
            pltpu.VMEM((C, TW), jnp.float32),
            pltpu.VMEM((C, TW), jnp.float32),
            pltpu.VMEM((C * ED,), jnp.float32),
            pltpu.VMEM((C * ED,), jnp.float32),
            pltpu.VMEM((ED, W), jnp.float32),
            pltpu.VMEM((W,), jnp.float32),
            pltpu.VMEM((WBC, TW), jnp.float32),
            pltpu.VMEM_SHARED((N, TW), jnp.float32),
            pltpu.SemaphoreType.DMA,
            pltpu.SemaphoreType.DMA,
            pltpu.SemaphoreType.DMA,
            pltpu.SemaphoreType.DMA,
            pltpu.SemaphoreType.DMA,
            pltpu.SemaphoreType.DMA,
            pltpu.SemaphoreType.DMA,
            pltpu.SemaphoreType.DMA,
        ],
    )
    def sc_agg(x_hbm, attr_hbm, wT_hbm, b_hbm, ei_hbm, zeros_hbm,
               out_hbm, src0_v, src1_v, dst0_v, dst1_v, xr0_v, xr1_v,
               av0_v, av1_v, wv_v, bv_v, wb_v, acc_sh,
               semA0, semA1, semT0, semT1, semG0, semG1, semS0, semS1):
        c = lax.axis_index("c")
        s = lax.axis_index("s")
        srcv = [src0_v, src1_v]
        dstv = [dst0_v, dst1_v]
        xrv = [xr0_v, xr1_v]
        avv = [av0_v, av1_v]
        semA = [semA0, semA1]
        semT = [semT0, semT1]
        semG = [semG0, semG1]
        semS = [semS0, semS1]
        pltpu.sync_copy(wT_hbm, wv_v)
        pltpu.sync_copy(b_hbm, bv_v)
        pltpu.sync_copy(zeros_hbm.at[pl.ds(0, RA)], acc_sh.at[pl.ds(s * RA, RA)])

        @pl.when(s == 15)
        def _zero_tail():
            pltpu.sync_copy(zeros_hbm.at[pl.ds(0, TAIL)],
                            acc_sh.at[pl.ds(16 * RA, TAIL)])

        plsc.subcore_barrier()

        base = (c * 16 + s) * EPW

        wsl = [[wv_v[k, pl.ds(16 * j, 16)] for k in range(ED)]
               for j in range(W // 16)]
        bsl = [bv_v[pl.ds(16 * j, 16)] for j in range(W // 16)]

        def _off(i):
            return pl.multiple_of(base + i * C, 8)

        def issueA(i, b):
            pltpu.async_copy(ei_hbm.at[pl.ds(_off(i), C)], srcv[b], semA[b])
            pltpu.async_copy(attr_hbm.at[pl.ds(_off(i) * ED, C * ED)],
                             avv[b], semA[b])

        def waitA(b):
            pltpu.make_async_copy(ei_hbm.at[pl.ds(0, C)], srcv[b],
                                  semA[b]).wait()
            pltpu.make_async_copy(attr_hbm.at[pl.ds(0, C * ED)], avv[b],
                                  semA[b]).wait()

        def issueT(i, b):
            pltpu.async_copy(ei_hbm.at[pl.ds(E + _off(i), C)], dstv[b],
                             semT[b])

        def waitT(b):
            pltpu.make_async_copy(ei_hbm.at[pl.ds(0, C)], dstv[b],
                                  semT[b]).wait()

        def issueG(b):
            pltpu.async_copy(x_hbm.at[srcv[b]], xrv[b], semG[b])

        def waitG(b):
            pltpu.make_async_copy(x_hbm.at[srcv[b]], xrv[b], semG[b]).wait()

        def issueS(b):
            pltpu.async_copy(xrv[b], acc_sh.at[dstv[b]], semS[b], add=True)

        def waitS(b):
            pltpu.make_async_copy(xrv[b], acc_sh.at[dstv[b]], semS[b]).wait()

        def compute(b):
            xr_v = xrv[b]
            av_v = avv[b]

            def grp(g, carry2):
                avec = av_v[pl.ds(g * 16, 16)]
                for t in range(4):
                    e = g * 4 + t
                    ab = [jnp.full((16,), avec[4 * t + k], jnp.float32)
                          for k in range(ED)]
                    for j in range(W // 16):
                        sl = pl.ds(16 * j, 16)
                        er = xr_v[e, sl] + bsl[j]
                        for k in range(ED):
                            er = er + ab[k] * wsl[j][k]
                        xr_v[e, sl] = jnp.maximum(er, 0.0)
                return carry2

            lax.fori_loop(0, C // 4, grp, 0)

        issueA(0, 0)
        issueT(0, 0)
        waitA(0)
        issueG(0)
        issueA(1, 1)

        def pair(m, carry):
            a = 2 * m
            waitA(1)

            @pl.when(m > 0)
            def _drain_s1():
                waitS(1)

            issueT(a + 1, 1)
            issueG(1)
            waitG(0)
            compute(0)
            waitT(0)
            issueS(0)
            issueA(a + 2, 0)
            waitG(1)
            compute(1)
            waitT(1)
            issueS(1)
            waitS(0)
            issueT(a + 2, 0)
            waitA(0)
            issueG(0)

            @pl.when(m < NCH // 2 - 1)
            def _prefetch_b2():
                issueA(a + 3, 1)

            return carry

        lax.fori_loop(0, NCH // 2, pair, 0)

        waitS(1)
        waitG(0)
        compute(0)
        waitT(0)
        issueS(0)
        waitS(0)
        plsc.subcore_barrier()

        for r in range(RA // WBC):
            row0 = s * RA + r * WBC
            pltpu.sync_copy(acc_sh.at[pl.ds(row0, WBC)], wb_v)
            pltpu.sync_copy(wb_v, out_hbm.at[c, pl.ds(row0, WBC)])

        @pl.when(s == 15)
        def _wb_tail():
            pltpu.sync_copy(acc_sh.at[pl.ds(16 * RA, TAIL)],
                            wb_v.at[pl.ds(0, TAIL)])
            pltpu.sync_copy(wb_v.at[pl.ds(0, TAIL)],
                            out_hbm.at[c, pl.ds(16 * RA, TAIL)])

    return sc_agg


_sc_agg128 = _make_sc_agg(F_IN, F_IN)
_sc_agg64 = _make_sc_agg(F_IN, H)


def _tc_mid(x, p0, p1, waT, ba, wbT, bb):
    BLK = 2000

    def body(x_ref, p0_ref, p1_ref, wa_ref, ba_ref, wb_ref, bb_ref,
             o_ref, oext_ref):
        h = x_ref[...] + p0_ref[...] + p1_ref[...]
        h = jnp.maximum(
            jnp.dot(h, wa_ref[...], preferred_element_type=jnp.float32)
            + ba_ref[...], 0.0)
        h = jnp.maximum(
            jnp.dot(h, wb_ref[...], preferred_element_type=jnp.float32)
            + bb_ref[...], 0.0)
        o_ref[...] = h
        oext_ref[...] = jnp.concatenate([h, jnp.zeros_like(h)], axis=1)

    return pl.pallas_call(
        body,
        grid=(N // BLK,),
        in_specs=[
            pl.BlockSpec((BLK, F_IN), lambda i: (i, 0)),
            pl.BlockSpec((BLK, F_IN), lambda i: (i, 0)),
            pl.BlockSpec((BLK, F_IN), lambda i: (i, 0)),
            pl.BlockSpec((F_IN, H), lambda i: (0, 0)),
            pl.BlockSpec((1, H), lambda i: (0, 0)),
            pl.BlockSpec((H, H), lambda i: (0, 0)),
            pl.BlockSpec((1, H), lambda i: (0, 0)),
        ],
        out_specs=[pl.BlockSpec((BLK, H), lambda i: (i, 0)),
                   pl.BlockSpec((BLK, F_IN), lambda i: (i, 0))],
        out_shape=[jax.ShapeDtypeStruct((N, H), jnp.float32),
                   jax.ShapeDtypeStruct((N, F_IN), jnp.float32)],
    )(x, p0, p1, waT, ba, wbT, bb)


def _tc_post(h1, p0, p1, batch2d, btot2d, w2aT, b2a, w2bT, b2b,
             rw1T, rb1, rw2T, rb2):
    def body(h_ref, p0_ref, p1_ref, bt_ref, bud_ref, wa_ref, ba_ref,
             wb_ref, bb_ref, r1_ref, c1_ref, r2_ref, c2_ref, o_ref):
        h = h_ref[...] + p0_ref[...] + p1_ref[...]
        h = jnp.maximum(
            jnp.dot(h, wa_ref[...], preferred_element_type=jnp.float32)
            + ba_ref[...], 0.0)
        h = jnp.maximum(
            jnp.dot(h, wb_ref[...], preferred_element_type=jnp.float32)
            + bb_ref[...], 0.0)
        z = jnp.maximum(
            jnp.dot(h, r1_ref[...], preferred_element_type=jnp.float32)
            + c1_ref[...], 0.0)
        slog = (jnp.dot(z, r2_ref[...], preferred_element_type=jnp.float32)
                + c2_ref[...])
        pi = 1.0 / (1.0 + jnp.exp(-slog))
        iota = lax.broadcasted_iota(jnp.int32, (N, G), 1)
        maskf = jnp.where(iota == bt_ref[...], 1.0, 0.0)
        totals = jnp.sum(pi * maskf, axis=0, keepdims=True)
        ratio = jnp.minimum(bud_ref[...] / (totals + 1e-12), 1.0)
        rn = jnp.sum(maskf * ratio, axis=1, keepdims=True)
        o_ref[...] = pi * rn

    return pl.pallas_call(
        body,
        out_shape=jax.ShapeDtypeStruct((N, 1), jnp.float32),
    )(h1, p0, p1, batch2d, btot2d, w2aT, b2a, w2bT, b2b, rw1T, rb1, rw2T, rb2)


def kernel(x, edge_index, edge_attr, batch, B_total, lin_e1_w, lin_e1_b,
           w1a, b1a, w1b, b1b, lin_e2_w, lin_e2_b, w2a, b2a, w2b, b2b,
           rw1, rb1, rw2, rb2):
    ei_flat = edge_index.reshape(-1)
    z128 = jnp.zeros((RA, F_IN), jnp.float32)

    attr_flat = edge_attr.reshape(-1)
    p1 = _sc_agg128(x, attr_flat, lin_e1_w.T, lin_e1_b, ei_flat, z128)
    h1, h1ext = _tc_mid(x, p1[0], p1[1], w1a.T, b1a.reshape(1, -1),
                        w1b.T, b1b.reshape(1, -1))
    p2 = _sc_agg64(h1ext, attr_flat, lin_e2_w.T, lin_e2_b, ei_flat, z128)
    out = _tc_post(h1, p2[0, :, :H], p2[1, :, :H],
                   batch.reshape(-1, 1).astype(jnp.int32),
                   B_total.reshape(1, -1), w2a.T, b2a.reshape(1, -1),
                   w2b.T, b2b.reshape(1, -1), rw1.T, rb1.reshape(1, -1),
                   rw2.T, rb2.reshape(1, -1))
    return out.reshape(-1)


# speedup vs baseline: 1.4531x; 1.4531x over previous
import functools

import jax
import jax.numpy as jnp
from jax import lax
from jax.experimental import pallas as pl
from jax.experimental.pallas import tpu as pltpu
from jax.experimental.pallas import tpu_sc as plsc

N, E, F_IN, H, ED, G = 10000, 320000, 128, 64, 4, 64

NW = 32
EPW = E // NW
C = 80
NCH = EPW // C
RA = 624
TAIL = N - 16 * RA
WBC = 208


def _make_sc_agg(TW, W):
    mesh = plsc.VectorSubcoreMesh(core_axis_name="c", subcore_axis_name="s")

    @functools.partial(
        pl.kernel,
        mesh=mesh,
        out_type=jax.ShapeDtypeStruct((2, N, TW), jnp.float32),
        scratch_types=[
            pltpu.VMEM((C,), jnp.int32),
            pltpu.VMEM((C,), jnp.int32),
            pltpu.VMEM((C,), jnp.int32),
            pltpu.VMEM((C,), jnp.int32),
            pltpu.VMEM((C, TW), jnp.float32),
            pltpu.VMEM((C, TW), jnp.float32),
            pltpu.VMEM((C * ED,), jnp.float32),
            pltpu.VMEM((C * ED,), jnp.float32),
            pltpu.VMEM((ED, W), jnp.float32),
            pltpu.VMEM((W,), jnp.float32),
            pltpu.VMEM((WBC, TW), jnp.float32),
            pltpu.VMEM_SHARED((N, TW), jnp.float32),
            pltpu.SemaphoreType.DMA,
            pltpu.SemaphoreType.DMA,
            pltpu.SemaphoreType.DMA,
            pltpu.SemaphoreType.DMA,
            pltpu.SemaphoreType.DMA,
            pltpu.SemaphoreType.DMA,
            pltpu.SemaphoreType.DMA,
            pltpu.SemaphoreType.DMA,
        ],
    )
    def sc_agg(x_hbm, attr_hbm, wT_hbm, b_hbm, ei_hbm, zeros_hbm,
               out_hbm, src0_v, src1_v, dst0_v, dst1_v, xr0_v, xr1_v,
               av0_v, av1_v, wv_v, bv_v, wb_v, acc_sh,
               semA0, semA1, semT0, semT1, semG0, semG1, semS0, semS1):
        c = lax.axis_index("c")
        s = lax.axis_index("s")
        srcv = [src0_v, src1_v]
        dstv = [dst0_v, dst1_v]
        xrv = [xr0_v, xr1_v]
        avv = [av0_v, av1_v]
        semA = [semA0, semA1]
        semT = [semT0, semT1]
        semG = [semG0, semG1]
        semS = [semS0, semS1]
        pltpu.sync_copy(wT_hbm, wv_v)
        pltpu.sync_copy(b_hbm, bv_v)
        pltpu.sync_copy(zeros_hbm.at[pl.ds(0, RA)], acc_sh.at[pl.ds(s * RA, RA)])

        @pl.when(s == 15)
        def _zero_tail():
            pltpu.sync_copy(zeros_hbm.at[pl.ds(0, TAIL)],
                            acc_sh.at[pl.ds(16 * RA, TAIL)])

        plsc.subcore_barrier()

        base = (c * 16 + s) * EPW

        wsl = [[wv_v[k, pl.ds(16 * j, 16)] for k in range(ED)]
               for j in range(W // 16)]
        bsl = [bv_v[pl.ds(16 * j, 16)] for j in range(W // 16)]

        def _off(i):
            return pl.multiple_of(base + i * C, 8)

        def issueA(i, b):
            pltpu.async_copy(ei_hbm.at[pl.ds(_off(i), C)], srcv[b], semA[b])
            pltpu.async_copy(attr_hbm.at[pl.ds(_off(i) * ED, C * ED)],
                             avv[b], semA[b])

        def waitA(b):
            pltpu.make_async_copy(ei_hbm.at[pl.ds(0, C)], srcv[b],
                                  semA[b]).wait()
            pltpu.make_async_copy(attr_hbm.at[pl.ds(0, C * ED)], avv[b],
                                  semA[b]).wait()

        def issueT(i, b):
            pltpu.async_copy(ei_hbm.at[pl.ds(E + _off(i), C)], dstv[b],
                             semT[b])

        def waitT(b):
            pltpu.make_async_copy(ei_hbm.at[pl.ds(0, C)], dstv[b],
                                  semT[b]).wait()

        def issueG(b):
            pltpu.async_copy(x_hbm.at[srcv[b]], xrv[b], semG[b])

        def waitG(b):
            pltpu.make_async_copy(x_hbm.at[srcv[b]], xrv[b], semG[b]).wait()

        def issueS(b):
            pltpu.async_copy(xrv[b], acc_sh.at[dstv[b]], semS[b], add=True)

        def waitS(b):
            pltpu.make_async_copy(xrv[b], acc_sh.at[dstv[b]], semS[b]).wait()

        def compute(b):
            xr_v = xrv[b]
            av_v = avv[b]

            def grp(g, carry2):
                avec = av_v[pl.ds(g * 16, 16)]
                for t in range(4):
                    e = g * 4 + t
                    ab = [jnp.full((16,), avec[4 * t + k], jnp.float32)
                          for k in range(ED)]
                    for j in range(W // 16):
                        sl = pl.ds(16 * j, 16)
                        er = bsl[j]
                        for k in range(ED):
                            er = er + ab[k] * wsl[j][k]
                        xr_v[e, sl] = jnp.maximum(xr_v[e, sl] + er, 0.0)
                return carry2

            lax.fori_loop(0, C // 4, grp, 0)

        issueA(0, 0)
        issueT(0, 0)
        waitA(0)
        issueG(0)
        issueA(1, 1)

        def pair(m, carry):
            a = 2 * m
            waitA(1)

            @pl.when(m > 0)
            def _drain_s1():
                waitS(1)

            issueT(a + 1, 1)
            issueG(1)
            waitG(0)
            compute(0)
            waitT(0)
            issueS(0)
            issueA(a + 2, 0)
            waitG(1)
            compute(1)
            waitT(1)
            issueS(1)
            waitS(0)
            issueT(a + 2, 0)
            waitA(0)
            issueG(0)

            @pl.when(m < NCH // 2 - 1)
            def _prefetch_b2():
                issueA(a + 3, 1)

            return carry

        lax.fori_loop(0, NCH // 2, pair, 0)

        waitS(1)
        waitG(0)
        compute(0)
        waitT(0)
        issueS(0)
        waitS(0)
        plsc.subcore_barrier()

        for r in range(RA // WBC):
            row0 = s * RA + r * WBC
            pltpu.sync_copy(acc_sh.at[pl.ds(row0, WBC)], wb_v)
            pltpu.sync_copy(wb_v, out_hbm.at[c, pl.ds(row0, WBC)])

        @pl.when(s == 15)
        def _wb_tail():
            pltpu.sync_copy(acc_sh.at[pl.ds(16 * RA, TAIL)],
                            wb_v.at[pl.ds(0, TAIL)])
            pltpu.sync_copy(wb_v.at[pl.ds(0, TAIL)],
                            out_hbm.at[c, pl.ds(16 * RA, TAIL)])

    return sc_agg


_sc_agg128 = _make_sc_agg(F_IN, F_IN)
_sc_agg64 = _make_sc_agg(F_IN, H)


def _tc_mid(x, p0, p1, waT, ba, wbT, bb):
    BLK = 2000

    def body(x_ref, p0_ref, p1_ref, wa_ref, ba_ref, wb_ref, bb_ref,
             o_ref, oext_ref):
        h = x_ref[...] + p0_ref[...] + p1_ref[...]
        h = jnp.maximum(
            jnp.dot(h, wa_ref[...], preferred_element_type=jnp.float32)
            + ba_ref[...], 0.0)
        h = jnp.maximum(
            jnp.dot(h, wb_ref[...], preferred_element_type=jnp.float32)
            + bb_ref[...], 0.0)
        o_ref[...] = h
        oext_ref[...] = jnp.concatenate([h, jnp.zeros_like(h)], axis=1)

    return pl.pallas_call(
        body,
        grid=(N // BLK,),
        in_specs=[
            pl.BlockSpec((BLK, F_IN), lambda i: (i, 0)),
            pl.BlockSpec((BLK, F_IN), lambda i: (i, 0)),
            pl.BlockSpec((BLK, F_IN), lambda i: (i, 0)),
            pl.BlockSpec((F_IN, H), lambda i: (0, 0)),
            pl.BlockSpec((1, H), lambda i: (0, 0)),
            pl.BlockSpec((H, H), lambda i: (0, 0)),
            pl.BlockSpec((1, H), lambda i: (0, 0)),
        ],
        out_specs=[pl.BlockSpec((BLK, H), lambda i: (i, 0)),
                   pl.BlockSpec((BLK, F_IN), lambda i: (i, 0))],
        out_shape=[jax.ShapeDtypeStruct((N, H), jnp.float32),
                   jax.ShapeDtypeStruct((N, F_IN), jnp.float32)],
    )(x, p0, p1, waT, ba, wbT, bb)


def _tc_post(h1, p0, p1, batch2d, btot2d, w2aT, b2a, w2bT, b2b,
             rw1T, rb1, rw2T, rb2):
    def body(h_ref, p0_ref, p1_ref, bt_ref, bud_ref, wa_ref, ba_ref,
             wb_ref, bb_ref, r1_ref, c1_ref, r2_ref, c2_ref, o_ref):
        h = h_ref[...] + p0_ref[...] + p1_ref[...]
        h = jnp.maximum(
            jnp.dot(h, wa_ref[...], preferred_element_type=jnp.float32)
            + ba_ref[...], 0.0)
        h = jnp.maximum(
            jnp.dot(h, wb_ref[...], preferred_element_type=jnp.float32)
            + bb_ref[...], 0.0)
        z = jnp.maximum(
            jnp.dot(h, r1_ref[...], preferred_element_type=jnp.float32)
            + c1_ref[...], 0.0)
        slog = (jnp.dot(z, r2_ref[...], preferred_element_type=jnp.float32)
                + c2_ref[...])
        pi = 1.0 / (1.0 + jnp.exp(-slog))
        iota = lax.broadcasted_iota(jnp.int32, (N, G), 1)
        maskf = jnp.where(iota == bt_ref[...], 1.0, 0.0)
        totals = jnp.sum(pi * maskf, axis=0, keepdims=True)
        ratio = jnp.minimum(bud_ref[...] / (totals + 1e-12), 1.0)
        rn = jnp.sum(maskf * ratio, axis=1, keepdims=True)
        o_ref[...] = pi * rn

    return pl.pallas_call(
        body,
        out_shape=jax.ShapeDtypeStruct((N, 1), jnp.float32),
    )(h1, p0, p1, batch2d, btot2d, w2aT, b2a, w2bT, b2b, rw1T, rb1, rw2T, rb2)


def kernel(x, edge_index, edge_attr, batch, B_total, lin_e1_w, lin_e1_b,
           w1a, b1a, w1b, b1b, lin_e2_w, lin_e2_b, w2a, b2a, w2b, b2b,
           rw1, rb1, rw2, rb2):
    ei_flat = edge_index.reshape(-1)
    z128 = jnp.zeros((RA, F_IN), jnp.float32)

    attr_flat = edge_attr.reshape(-1)
    p1 = _sc_agg128(x, attr_flat, lin_e1_w.T, lin_e1_b, ei_flat, z128)
    h1, h1ext = _tc_mid(x, p1[0], p1[1], w1a.T, b1a.reshape(1, -1),
                        w1b.T, b1b.reshape(1, -1))
    p2 = _sc_agg64(h1ext, attr_flat, lin_e2_w.T, lin_e2_b, ei_flat, z128)
    out = _tc_post(h1, p2[0, :, :H], p2[1, :, :H],
                   batch.reshape(-1, 1).astype(jnp.int32),
                   B_total.reshape(1, -1), w2a.T, b2a.reshape(1, -1),
                   w2b.T, b2b.reshape(1, -1), rw1.T, rb1.reshape(1, -1),
                   rw2.T, rb2.reshape(1, -1))
    return out.reshape(-1)


# speedup vs baseline: 1.4833x; 1.0208x over previous
import functools

import jax
import jax.numpy as jnp
from jax import lax
from jax.experimental import pallas as pl
from jax.experimental.pallas import tpu as pltpu
from jax.experimental.pallas import tpu_sc as plsc

N, E, F_IN, H, ED, G = 10000, 320000, 128, 64, 4, 64

NW = 32
C = 128
T_CH = E // C
CPW = T_CH // NW
NTAIL = T_CH - CPW * NW
RA = 624
TAIL = N - 16 * RA
WBC = 48


def _make_sc_agg(TW, W):
    mesh = plsc.VectorSubcoreMesh(core_axis_name="c", subcore_axis_name="s")

    @functools.partial(
        pl.kernel,
        mesh=mesh,
        out_type=jax.ShapeDtypeStruct((2, N, TW), jnp.float32),
        scratch_types=[
            pltpu.VMEM((C,), jnp.int32),
            pltpu.VMEM((C,), jnp.int32),
            pltpu.VMEM((C,), jnp.int32),
            pltpu.VMEM((C,), jnp.int32),
            pltpu.VMEM((C, TW), jnp.float32),
            pltpu.VMEM((C, TW), jnp.float32),
            pltpu.VMEM((C * ED,), jnp.float32),
            pltpu.VMEM((C * ED,), jnp.float32),
            pltpu.VMEM((ED, W), jnp.float32),
            pltpu.VMEM((W,), jnp.float32),
            pltpu.VMEM((WBC, TW), jnp.float32),
            pltpu.VMEM_SHARED((N, TW), jnp.float32),
            pltpu.SemaphoreType.DMA,
            pltpu.SemaphoreType.DMA,
            pltpu.SemaphoreType.DMA,
            pltpu.SemaphoreType.DMA,
            pltpu.SemaphoreType.DMA,
            pltpu.SemaphoreType.DMA,
            pltpu.SemaphoreType.DMA,
            pltpu.SemaphoreType.DMA,
        ],
    )
    def sc_agg(x_hbm, attr_hbm, wT_hbm, b_hbm, ei_hbm, zeros_hbm,
               out_hbm, src0_v, src1_v, dst0_v, dst1_v, xr0_v, xr1_v,
               av0_v, av1_v, wv_v, bv_v, wb_v, acc_sh,
               semA0, semA1, semT0, semT1, semG0, semG1, semS0, semS1):
        c = lax.axis_index("c")
        s = lax.axis_index("s")
        srcv = [src0_v, src1_v]
        dstv = [dst0_v, dst1_v]
        xrv = [xr0_v, xr1_v]
        avv = [av0_v, av1_v]
        semA = [semA0, semA1]
        semT = [semT0, semT1]
        semG = [semG0, semG1]
        semS = [semS0, semS1]
        pltpu.sync_copy(wT_hbm, wv_v)
        pltpu.sync_copy(b_hbm, bv_v)
        pltpu.sync_copy(zeros_hbm.at[pl.ds(0, RA)], acc_sh.at[pl.ds(s * RA, RA)])

        @pl.when(s == 15)
        def _zero_tail():
            pltpu.sync_copy(zeros_hbm.at[pl.ds(0, TAIL)],
                            acc_sh.at[pl.ds(16 * RA, TAIL)])

        plsc.subcore_barrier()

        wid = c * 16 + s
        base = wid * CPW

        wsl = [[wv_v[k, pl.ds(16 * j, 16)] for k in range(ED)]
               for j in range(W // 16)]
        bsl = [bv_v[pl.ds(16 * j, 16)] for j in range(W // 16)]

        def _off(i):
            return pl.multiple_of((base + i) * C, 8)

        def issueA(i, b):
            pltpu.async_copy(ei_hbm.at[pl.ds(_off(i), C)], srcv[b], semA[b])
            pltpu.async_copy(attr_hbm.at[pl.ds(_off(i) * ED, C * ED)],
                             avv[b], semA[b])

        def waitA(b):
            pltpu.make_async_copy(ei_hbm.at[pl.ds(0, C)], srcv[b],
                                  semA[b]).wait()
            pltpu.make_async_copy(attr_hbm.at[pl.ds(0, C * ED)], avv[b],
                                  semA[b]).wait()

        def issueT(i, b):
            pltpu.async_copy(ei_hbm.at[pl.ds(E + _off(i), C)], dstv[b],
                             semT[b])

        def waitT(b):
            pltpu.make_async_copy(ei_hbm.at[pl.ds(0, C)], dstv[b],
                                  semT[b]).wait()

        def issueG(b):
            pltpu.async_copy(x_hbm.at[srcv[b]], xrv[b], semG[b])

        def waitG(b):
            pltpu.make_async_copy(x_hbm.at[srcv[b]], xrv[b], semG[b]).wait()

        def issueS(b):
            pltpu.async_copy(xrv[b], acc_sh.at[dstv[b]], semS[b], add=True)

        def waitS(b):
            pltpu.make_async_copy(xrv[b], acc_sh.at[dstv[b]], semS[b]).wait()

        def compute(b):
            xr_v = xrv[b]
            av_v = avv[b]

            def grp(g, carry2):
                avec = av_v[pl.ds(g * 16, 16)]
                for t in range(4):
                    e = g * 4 + t
                    ab = [jnp.full((16,), avec[4 * t + k], jnp.float32)
                          for k in range(ED)]
                    for j in range(W // 16):
                        sl = pl.ds(16 * j, 16)
                        er = bsl[j]
                        for k in range(ED):
                            er = er + ab[k] * wsl[j][k]
                        xr_v[e, sl] = jnp.maximum(xr_v[e, sl] + er, 0.0)
                return carry2

            lax.fori_loop(0, C // 4, grp, 0)

        NPAIR = CPW // 2
        issueA(0, 0)
        issueT(0, 0)
        waitA(0)
        issueG(0)
        issueA(1, 1)

        def pair(m, carry):
            a = 2 * m
            waitA(1)

            @pl.when(m > 0)
            def _drain_s1():
                waitS(1)

            issueT(a + 1, 1)
            issueG(1)
            waitG(0)
            compute(0)
            waitT(0)
            issueS(0)

            @pl.when(m < NPAIR - 1)
            def _prefetch_a2():
                issueA(a + 2, 0)

            waitG(1)
            compute(1)
            waitT(1)
            issueS(1)
            waitS(0)

            @pl.when(m < NPAIR - 1)
            def _next_g0():
                issueT(a + 2, 0)
                waitA(0)
                issueG(0)
                issueA(a + 3, 1)

            return carry

        lax.fori_loop(0, NPAIR, pair, 0)
        waitS(1)

        @pl.when(wid < NTAIL)
        def _tail_chunk():
            i = NW * CPW - base + wid
            issueA(i, 0)
            issueT(i, 0)
            waitA(0)
            issueG(0)
            waitG(0)
            compute(0)
            waitT(0)
            issueS(0)
            waitS(0)

        plsc.subcore_barrier()

        for r in range(RA // WBC):
            row0 = s * RA + r * WBC
            pltpu.sync_copy(acc_sh.at[pl.ds(row0, WBC)], wb_v)
            pltpu.sync_copy(wb_v, out_hbm.at[c, pl.ds(row0, WBC)])

        @pl.when(s == 15)
        def _wb_tail():
            pltpu.sync_copy(acc_sh.at[pl.ds(16 * RA, TAIL)],
                            wb_v.at[pl.ds(0, TAIL)])
            pltpu.sync_copy(wb_v.at[pl.ds(0, TAIL)],
                            out_hbm.at[c, pl.ds(16 * RA, TAIL)])

    return sc_agg


_sc_agg128 = _make_sc_agg(F_IN, F_IN)
_sc_agg64 = _make_sc_agg(F_IN, H)


def _tc_mid(x, p0, p1, waT, ba, wbT, bb):
    BLK = 2000

    def body(x_ref, p0_ref, p1_ref, wa_ref, ba_ref, wb_ref, bb_ref,
             o_ref, oext_ref):
        h = x_ref[...] + p0_ref[...] + p1_ref[...]
        h = jnp.maximum(
            jnp.dot(h, wa_ref[...], preferred_element_type=jnp.float32)
            + ba_ref[...], 0.0)
        h = jnp.maximum(
            jnp.dot(h, wb_ref[...], preferred_element_type=jnp.float32)
            + bb_ref[...], 0.0)
        o_ref[...] = h
        oext_ref[...] = jnp.concatenate([h, jnp.zeros_like(h)], axis=1)

    return pl.pallas_call(
        body,
        grid=(N // BLK,),
        in_specs=[
            pl.BlockSpec((BLK, F_IN), lambda i: (i, 0)),
            pl.BlockSpec((BLK, F_IN), lambda i: (i, 0)),
            pl.BlockSpec((BLK, F_IN), lambda i: (i, 0)),
            pl.BlockSpec((F_IN, H), lambda i: (0, 0)),
            pl.BlockSpec((1, H), lambda i: (0, 0)),
            pl.BlockSpec((H, H), lambda i: (0, 0)),
            pl.BlockSpec((1, H), lambda i: (0, 0)),
        ],
        out_specs=[pl.BlockSpec((BLK, H), lambda i: (i, 0)),
                   pl.BlockSpec((BLK, F_IN), lambda i: (i, 0))],
        out_shape=[jax.ShapeDtypeStruct((N, H), jnp.float32),
                   jax.ShapeDtypeStruct((N, F_IN), jnp.float32)],
    )(x, p0, p1, waT, ba, wbT, bb)


def _tc_post(h1, p0, p1, batch2d, btot2d, w2aT, b2a, w2bT, b2b,
             rw1T, rb1, rw2T, rb2):
    def body(h_ref, p0_ref, p1_ref, bt_ref, bud_ref, wa_ref, ba_ref,
             wb_ref, bb_ref, r1_ref, c1_ref, r2_ref, c2_ref, o_ref):
        h = h_ref[...] + p0_ref[...] + p1_ref[...]
        h = jnp.maximum(
            jnp.dot(h, wa_ref[...], preferred_element_type=jnp.float32)
            + ba_ref[...], 0.0)
        h = jnp.maximum(
            jnp.dot(h, wb_ref[...], preferred_element_type=jnp.float32)
            + bb_ref[...], 0.0)
        z = jnp.maximum(
            jnp.dot(h, r1_ref[...], preferred_element_type=jnp.float32)
            + c1_ref[...], 0.0)
        slog = (jnp.dot(z, r2_ref[...], preferred_element_type=jnp.float32)
                + c2_ref[...])
        pi = 1.0 / (1.0 + jnp.exp(-slog))
        iota = lax.broadcasted_iota(jnp.int32, (N, G), 1)
        maskf = jnp.where(iota == bt_ref[...], 1.0, 0.0)
        totals = jnp.sum(pi * maskf, axis=0, keepdims=True)
        ratio = jnp.minimum(bud_ref[...] / (totals + 1e-12), 1.0)
        rn = jnp.sum(maskf * ratio, axis=1, keepdims=True)
        o_ref[...] = pi * rn

    return pl.pallas_call(
        body,
        out_shape=jax.ShapeDtypeStruct((N, 1), jnp.float32),
    )(h1, p0, p1, batch2d, btot2d, w2aT, b2a, w2bT, b2b, rw1T, rb1, rw2T, rb2)


def kernel(x, edge_index, edge_attr, batch, B_total, lin_e1_w, lin_e1_b,
           w1a, b1a, w1b, b1b, lin_e2_w, lin_e2_b, w2a, b2a, w2b, b2b,
           rw1, rb1, rw2, rb2):
    ei_flat = edge_index.reshape(-1)
    z128 = jnp.zeros((RA, F_IN), jnp.float32)

    attr_flat = edge_attr.reshape(-1)
    p1 = _sc_agg128(x, attr_flat, lin_e1_w.T, lin_e1_b, ei_flat, z128)
    h1, h1ext = _tc_mid(x, p1[0], p1[1], w1a.T, b1a.reshape(1, -1),
                        w1b.T, b1b.reshape(1, -1))
    p2 = _sc_agg64(h1ext, attr_flat, lin_e2_w.T, lin_e2_b, ei_flat, z128)
    out = _tc_post(h1, p2[0, :, :H], p2[1, :, :H],
                   batch.reshape(-1, 1).astype(jnp.int32),
                   B_total.reshape(1, -1), w2a.T, b2a.reshape(1, -1),
                   w2b.T, b2b.reshape(1, -1), rw1.T, rb1.reshape(1, -1),
                   rw2.T, rb2.reshape(1, -1))
    return out.reshape(-1)


# speedup vs baseline: 1.4900x; 1.0045x over previous
import functools

import jax
import jax.numpy as jnp
from jax import lax
from jax.experimental import pallas as pl
from jax.experimental.pallas import tpu as pltpu
from jax.experimental.pallas import tpu_sc as plsc

N, E, F_IN, H, ED, G = 10000, 320000, 128, 64, 4, 64

NW = 32
C = 128
T_CH = E // C
CPW = T_CH // NW
NTAIL = T_CH - CPW * NW
RA = 624
TAIL = N - 16 * RA
WBC = 48


def _make_sc_agg(TW, W):
    mesh = plsc.VectorSubcoreMesh(core_axis_name="c", subcore_axis_name="s")

    @functools.partial(
        pl.kernel,
        mesh=mesh,
        out_type=jax.ShapeDtypeStruct((2, N, TW), jnp.float32),
        scratch_types=[
            pltpu.VMEM((C,), jnp.int32),
            pltpu.VMEM((C,), jnp.int32),
            pltpu.VMEM((C,), jnp.int32),
            pltpu.VMEM((C,), jnp.int32),
            pltpu.VMEM((C, TW), jnp.float32),
            pltpu.VMEM((C, TW), jnp.float32),
            pltpu.VMEM((C * ED,), jnp.float32),
            pltpu.VMEM((C * ED,), jnp.float32),
            pltpu.VMEM((ED, W), jnp.float32),
            pltpu.VMEM((W,), jnp.float32),
            pltpu.VMEM((WBC, TW), jnp.float32),
            pltpu.VMEM_SHARED((N, TW), jnp.float32),
            pltpu.SemaphoreType.DMA,
            pltpu.SemaphoreType.DMA,
            pltpu.SemaphoreType.DMA,
            pltpu.SemaphoreType.DMA,
            pltpu.SemaphoreType.DMA,
            pltpu.SemaphoreType.DMA,
            pltpu.SemaphoreType.DMA,
            pltpu.SemaphoreType.DMA,
        ],
    )
    def sc_agg(x_hbm, attr_hbm, wT_hbm, b_hbm, ei_hbm, zeros_hbm,
               out_hbm, src0_v, src1_v, dst0_v, dst1_v, xr0_v, xr1_v,
               av0_v, av1_v, wv_v, bv_v, wb_v, acc_sh,
               semA0, semA1, semT0, semT1, semG0, semG1, semS0, semS1):
        c = lax.axis_index("c")
        s = lax.axis_index("s")
        srcv = [src0_v, src1_v]
        dstv = [dst0_v, dst1_v]
        xrv = [xr0_v, xr1_v]
        avv = [av0_v, av1_v]
        semA = [semA0, semA1]
        semT = [semT0, semT1]
        semG = [semG0, semG1]
        semS = [semS0, semS1]
        pltpu.sync_copy(wT_hbm, wv_v)
        pltpu.sync_copy(b_hbm, bv_v)
        wid = c * 16 + s
        base = wid * CPW

        wsl = [[wv_v[k, pl.ds(16 * j, 16)] for k in range(ED)]
               for j in range(W // 16)]
        bsl = [bv_v[pl.ds(16 * j, 16)] for j in range(W // 16)]

        def _off(i):
            return pl.multiple_of((base + i) * C, 8)

        def issueA(i, b):
            pltpu.async_copy(ei_hbm.at[pl.ds(_off(i), C)], srcv[b], semA[b])
            pltpu.async_copy(attr_hbm.at[pl.ds(_off(i) * ED, C * ED)],
                             avv[b], semA[b])

        def waitA(b):
            pltpu.make_async_copy(ei_hbm.at[pl.ds(0, C)], srcv[b],
                                  semA[b]).wait()
            pltpu.make_async_copy(attr_hbm.at[pl.ds(0, C * ED)], avv[b],
                                  semA[b]).wait()

        def issueT(i, b):
            pltpu.async_copy(ei_hbm.at[pl.ds(E + _off(i), C)], dstv[b],
                             semT[b])

        def waitT(b):
            pltpu.make_async_copy(ei_hbm.at[pl.ds(0, C)], dstv[b],
                                  semT[b]).wait()

        def issueG(b):
            pltpu.async_copy(x_hbm.at[srcv[b]], xrv[b], semG[b])

        def waitG(b):
            pltpu.make_async_copy(x_hbm.at[srcv[b]], xrv[b], semG[b]).wait()

        def issueS(b):
            pltpu.async_copy(xrv[b], acc_sh.at[dstv[b]], semS[b], add=True)

        def waitS(b):
            pltpu.make_async_copy(xrv[b], acc_sh.at[dstv[b]], semS[b]).wait()

        def compute(b):
            xr_v = xrv[b]
            av_v = avv[b]

            def grp(g, carry2):
                avec = av_v[pl.ds(g * 16, 16)]
                for t in range(4):
                    e = g * 4 + t
                    ab = [jnp.full((16,), avec[4 * t + k], jnp.float32)
                          for k in range(ED)]
                    for j in range(W // 16):
                        sl = pl.ds(16 * j, 16)
                        er = bsl[j]
                        for k in range(ED):
                            er = er + ab[k] * wsl[j][k]
                        xr_v[e, sl] = jnp.maximum(xr_v[e, sl] + er, 0.0)
                return carry2

            lax.fori_loop(0, C // 4, grp, 0)

        NPAIR = CPW // 2
        issueA(0, 0)
        issueT(0, 0)
        issueA(1, 1)

        pltpu.sync_copy(zeros_hbm.at[pl.ds(0, RA)], acc_sh.at[pl.ds(s * RA, RA)])

        @pl.when(s == 15)
        def _zero_tail():
            pltpu.sync_copy(zeros_hbm.at[pl.ds(0, TAIL)],
                            acc_sh.at[pl.ds(16 * RA, TAIL)])

        waitA(0)
        issueG(0)
        plsc.subcore_barrier()

        def pair(m, carry):
            a = 2 * m
            waitA(1)

            @pl.when(m > 0)
            def _drain_s1():
                waitS(1)

            issueT(a + 1, 1)
            issueG(1)
            waitG(0)
            compute(0)
            waitT(0)
            issueS(0)

            @pl.when(m < NPAIR - 1)
            def _prefetch_a2():
                issueA(a + 2, 0)

            waitG(1)
            compute(1)
            waitT(1)
            issueS(1)
            waitS(0)

            @pl.when(m < NPAIR - 1)
            def _next_g0():
                issueT(a + 2, 0)
                waitA(0)
                issueG(0)
                issueA(a + 3, 1)

            return carry

        lax.fori_loop(0, NPAIR, pair, 0)
        waitS(1)

        @pl.when(wid < NTAIL)
        def _tail_chunk():
            i = NW * CPW - base + wid
            issueA(i, 0)
            issueT(i, 0)
            waitA(0)
            issueG(0)
            waitG(0)
            compute(0)
            waitT(0)
            issueS(0)
            waitS(0)

        plsc.subcore_barrier()

        for r in range(RA // WBC):
            row0 = s * RA + r * WBC
            pltpu.sync_copy(acc_sh.at[pl.ds(row0, WBC)], wb_v)
            pltpu.sync_copy(wb_v, out_hbm.at[c, pl.ds(row0, WBC)])

        @pl.when(s == 15)
        def _wb_tail():
            pltpu.sync_copy(acc_sh.at[pl.ds(16 * RA, TAIL)],
                            wb_v.at[pl.ds(0, TAIL)])
            pltpu.sync_copy(wb_v.at[pl.ds(0, TAIL)],
                            out_hbm.at[c, pl.ds(16 * RA, TAIL)])

    return sc_agg


_sc_agg128 = _make_sc_agg(F_IN, F_IN)
_sc_agg64 = _make_sc_agg(F_IN, H)


def _tc_mid(x, p0, p1, waT, ba, wbT, bb):
    BLK = 2000

    def body(x_ref, p0_ref, p1_ref, wa_ref, ba_ref, wb_ref, bb_ref,
             o_ref, oext_ref):
        h = x_ref[...] + p0_ref[...] + p1_ref[...]
        h = jnp.maximum(
            jnp.dot(h, wa_ref[...], preferred_element_type=jnp.float32)
            + ba_ref[...], 0.0)
        h = jnp.maximum(
            jnp.dot(h, wb_ref[...], preferred_element_type=jnp.float32)
            + bb_ref[...], 0.0)
        o_ref[...] = h
        oext_ref[...] = jnp.concatenate([h, jnp.zeros_like(h)], axis=1)

    return pl.pallas_call(
        body,
        grid=(N // BLK,),
        in_specs=[
            pl.BlockSpec((BLK, F_IN), lambda i: (i, 0)),
            pl.BlockSpec((BLK, F_IN), lambda i: (i, 0)),
            pl.BlockSpec((BLK, F_IN), lambda i: (i, 0)),
            pl.BlockSpec((F_IN, H), lambda i: (0, 0)),
            pl.BlockSpec((1, H), lambda i: (0, 0)),
            pl.BlockSpec((H, H), lambda i: (0, 0)),
            pl.BlockSpec((1, H), lambda i: (0, 0)),
        ],
        out_specs=[pl.BlockSpec((BLK, H), lambda i: (i, 0)),
                   pl.BlockSpec((BLK, F_IN), lambda i: (i, 0))],
        out_shape=[jax.ShapeDtypeStruct((N, H), jnp.float32),
                   jax.ShapeDtypeStruct((N, F_IN), jnp.float32)],
    )(x, p0, p1, waT, ba, wbT, bb)


def _tc_post(h1, p0, p1, batch2d, btot2d, w2aT, b2a, w2bT, b2b,
             rw1T, rb1, rw2T, rb2):
    def body(h_ref, p0_ref, p1_ref, bt_ref, bud_ref, wa_ref, ba_ref,
             wb_ref, bb_ref, r1_ref, c1_ref, r2_ref, c2_ref, o_ref):
        h = h_ref[...] + p0_ref[...] + p1_ref[...]
        h = jnp.maximum(
            jnp.dot(h, wa_ref[...], preferred_element_type=jnp.float32)
            + ba_ref[...], 0.0)
        h = jnp.maximum(
            jnp.dot(h, wb_ref[...], preferred_element_type=jnp.float32)
            + bb_ref[...], 0.0)
        z = jnp.maximum(
            jnp.dot(h, r1_ref[...], preferred_element_type=jnp.float32)
            + c1_ref[...], 0.0)
        slog = (jnp.dot(z, r2_ref[...], preferred_element_type=jnp.float32)
                + c2_ref[...])
        pi = 1.0 / (1.0 + jnp.exp(-slog))
        iota = lax.broadcasted_iota(jnp.int32, (N, G), 1)
        maskf = jnp.where(iota == bt_ref[...], 1.0, 0.0)
        totals = jnp.sum(pi * maskf, axis=0, keepdims=True)
        ratio = jnp.minimum(bud_ref[...] / (totals + 1e-12), 1.0)
        rn = jnp.sum(maskf * ratio, axis=1, keepdims=True)
        o_ref[...] = pi * rn

    return pl.pallas_call(
        body,
        out_shape=jax.ShapeDtypeStruct((N, 1), jnp.float32),
    )(h1, p0, p1, batch2d, btot2d, w2aT, b2a, w2bT, b2b, rw1T, rb1, rw2T, rb2)


def kernel(x, edge_index, edge_attr, batch, B_total, lin_e1_w, lin_e1_b,
           w1a, b1a, w1b, b1b, lin_e2_w, lin_e2_b, w2a, b2a, w2b, b2b,
           rw1, rb1, rw2, rb2):
    ei_flat = edge_index.reshape(-1)
    z128 = jnp.zeros((RA, F_IN), jnp.float32)

    attr_flat = edge_attr.reshape(-1)
    p1 = _sc_agg128(x, attr_flat, lin_e1_w.T, lin_e1_b, ei_flat, z128)
    h1, h1ext = _tc_mid(x, p1[0], p1[1], w1a.T, b1a.reshape(1, -1),
                        w1b.T, b1b.reshape(1, -1))
    p2 = _sc_agg64(h1ext, attr_flat, lin_e2_w.T, lin_e2_b, ei_flat, z128)
    out = _tc_post(h1, p2[0, :, :H], p2[1, :, :H],
                   batch.reshape(-1, 1).astype(jnp.int32),
                   B_total.reshape(1, -1), w2a.T, b2a.reshape(1, -1),
                   w2b.T, b2b.reshape(1, -1), rw1.T, rb1.reshape(1, -1),
                   rw2.T, rb2.reshape(1, -1))
    return out.reshape(-1)
